# baseline (speedup 1.0000x reference)
import functools

import jax
import jax.numpy as jnp
from jax import lax
from jax.experimental import pallas as pl
from jax.experimental.pallas import tpu as pltpu
from jax.experimental.pallas import tpu_sc as plsc

VOCAB = 100000
EMB_DIM = 16
BATCH = 16384
HIST = 200

NC = 2
NS = 16
NW = NC * NS
ROWS_PER = BATCH // NW
CHUNK = 64
LANES = 16


def _tc_table_w(table_r, m, b2):
    def body(tbl_ref, m_ref, b_ref, out_ref):
        out_ref[...] = (
            jnp.dot(tbl_ref[...], m_ref[...], preferred_element_type=jnp.float32)
            + b_ref[0, 0]
        )

    return pl.pallas_call(
        body,
        out_shape=jax.ShapeDtypeStruct((VOCAB // 8, 8), jnp.float32),
        in_specs=[
            pl.BlockSpec(memory_space=pltpu.VMEM),
            pl.BlockSpec(memory_space=pltpu.VMEM),
            pl.BlockSpec(memory_space=pltpu.SMEM),
        ],
        out_specs=pl.BlockSpec(memory_space=pltpu.VMEM),
    )(table_r, m, b2)


def _sc_gather_sum(t, x_flat):
    mesh = plsc.VectorSubcoreMesh(
        core_axis_name="c", subcore_axis_name="s", num_cores=NC, num_subcores=NS
    )

    @functools.partial(
        pl.kernel,
        out_type=jax.ShapeDtypeStruct((BATCH,), jnp.float32),
        mesh=mesh,
        compiler_params=pltpu.CompilerParams(
            needs_layout_passes=False, use_tc_tiling_on_sc=False
        ),
        scratch_types=[
            pltpu.VMEM((VOCAB,), jnp.float32),
            pltpu.VMEM((CHUNK * HIST // 128, 128), jnp.int32),
            pltpu.VMEM((CHUNK * HIST // 128, 128), jnp.int32),
            pltpu.VMEM((ROWS_PER,), jnp.float32),
            pltpu.SemaphoreType.DMA,
            pltpu.SemaphoreType.DMA,
            pltpu.SemaphoreType.DMA,
        ],
    )
    def k(t_hbm, x_hbm, out_hbm, t_v, x_v0, x_v1, s_v, sem_t, sem0, sem1):
        wid = lax.axis_index("c") * NS + lax.axis_index("s")
        base = wid * ROWS_PER
        bufs = (x_v0, x_v1)
        sems = (sem0, sem1)
        n_chunks = ROWS_PER // CHUNK

        CROWS = CHUNK * HIST // 128
        xrow0 = base * HIST // 128

        t_cp = pltpu.async_copy(t_hbm, t_v, sem_t)
        cps = [None, None]
        cps[0] = pltpu.async_copy(x_hbm.at[pl.ds(xrow0, CROWS)], x_v0, sem0)
        t_cp.wait()
        iota = lax.iota(jnp.int32, LANES)
        UNR = 8
        for c in range(n_chunks):
            if c + 1 < n_chunks:
                nxt = (c + 1) % 2
                cps[nxt] = pltpu.async_copy(
                    x_hbm.at[pl.ds(xrow0 + (c + 1) * CROWS, CROWS)],
                    bufs[nxt],
                    sems[nxt],
                )
            cps[c % 2].wait()
            x_v = bufs[c % 2]
            for g in range(0, CHUNK // LANES, 2):
                offs_a = [(iota + g * LANES) * HIST + k for k in range(UNR)]
                offs_b = [(iota + (g + 1) * LANES) * HIST + k for k in range(UNR)]

                def jbody(i, carry, offs_a=offs_a, offs_b=offs_b, x_v=x_v):
                    acc_a, acc_b = carry
                    j = i * UNR
                    va = []
                    vb = []
                    for k in range(UNR):
                        pa = offs_a[k] + j
                        pb = offs_b[k] + j
                        ia = plsc.load_gather(x_v, [pa >> 7, pa & 127])
                        ib = plsc.load_gather(x_v, [pb >> 7, pb & 127])
                        va.append(plsc.load_gather(t_v, [ia]))
                        vb.append(plsc.load_gather(t_v, [ib]))
                    sa = ((va[0] + va[1]) + (va[2] + va[3])) + (
                        (va[4] + va[5]) + (va[6] + va[7])
                    )
                    sb = ((vb[0] + vb[1]) + (vb[2] + vb[3])) + (
                        (vb[4] + vb[5]) + (vb[6] + vb[7])
                    )
                    return acc_a + sa, acc_b + sb

                zero = jnp.zeros((LANES,), jnp.float32)
                acc_a, acc_b = lax.fori_loop(0, HIST // UNR, jbody, (zero, zero))
                s_v[pl.ds(c * CHUNK + g * LANES, LANES)] = acc_a
                s_v[pl.ds(c * CHUNK + (g + 1) * LANES, LANES)] = acc_b
        for i in range(ROWS_PER // LANES):
            sv = s_v[pl.ds(i * LANES, LANES)]
            p = 1.0 / (1.0 + jnp.exp(sv * (-1.0 / HIST)))
            yi = (p * 10000.0 + 0.5).astype(jnp.int32)
            s_v[pl.ds(i * LANES, LANES)] = yi.astype(jnp.float32) * (1.0 / 10000.0)
        pltpu.sync_copy(s_v, out_hbm.at[pl.ds(base, ROWS_PER)])

    return k(t, x_flat)


def kernel(x, table, W, b):
    m = jnp.kron(jnp.eye(8, dtype=jnp.float32), W.astype(jnp.float32))
    table_r = table.reshape(VOCAB // 8, 128)
    b2 = b.reshape(1, 1)

    t = _tc_table_w(table_r, m, b2).reshape(VOCAB)
    t, x_gate = lax.optimization_barrier((t, x))
    probs = _sc_gather_sum(t, x_gate.reshape(BATCH * HIST // 128, 128))
    return probs.reshape(BATCH, 1)


# speedup vs baseline: 1.0717x; 1.0717x over previous
import functools

import jax
import jax.numpy as jnp
from jax import lax
from jax.experimental import pallas as pl
from jax.experimental.pallas import tpu as pltpu
from jax.experimental.pallas import tpu_sc as plsc

VOCAB = 100000
EMB_DIM = 16
BATCH = 16384
HIST = 200

NC = 2
NS = 16
NW = NC * NS
ROWS_PER = BATCH // NW
CHUNK = 64
LANES = 16


def _tc_table_w(table_r, m, b2):
    def body(tbl_ref, m_ref, b_ref, out_ref):
        out_ref[...] = (
            jnp.dot(tbl_ref[...], m_ref[...], preferred_element_type=jnp.float32)
            + b_ref[0, 0]
        )

    return pl.pallas_call(
        body,
        out_shape=jax.ShapeDtypeStruct((VOCAB // 8, 8), jnp.float32),
        in_specs=[
            pl.BlockSpec(memory_space=pltpu.VMEM),
            pl.BlockSpec(memory_space=pltpu.VMEM),
            pl.BlockSpec(memory_space=pltpu.SMEM),
        ],
        out_specs=pl.BlockSpec(memory_space=pltpu.VMEM),
    )(table_r, m, b2)


def _sc_gather_sum(t, x_flat):
    mesh = plsc.VectorSubcoreMesh(
        core_axis_name="c", subcore_axis_name="s", num_cores=NC, num_subcores=NS
    )

    @functools.partial(
        pl.kernel,
        out_type=jax.ShapeDtypeStruct((BATCH,), jnp.float32),
        mesh=mesh,
        compiler_params=pltpu.CompilerParams(
            needs_layout_passes=False, use_tc_tiling_on_sc=False
        ),
        scratch_types=[
            pltpu.VMEM((VOCAB,), jnp.float32),
            pltpu.VMEM((CHUNK * HIST // 128, 128), jnp.int32),
            pltpu.VMEM((CHUNK * HIST // 128, 128), jnp.int32),
            pltpu.VMEM((ROWS_PER,), jnp.float32),
            pltpu.SemaphoreType.DMA,
            pltpu.SemaphoreType.DMA,
            pltpu.SemaphoreType.DMA,
        ],
    )
    def k(t_hbm, x_hbm, out_hbm, t_v, x_v0, x_v1, s_v, sem_t, sem0, sem1):
        wid = lax.axis_index("c") * NS + lax.axis_index("s")
        base = wid * ROWS_PER
        bufs = (x_v0, x_v1)
        sems = (sem0, sem1)
        n_chunks = ROWS_PER // CHUNK

        CROWS = CHUNK * HIST // 128
        xrow0 = base * HIST // 128

        t_cp = pltpu.async_copy(t_hbm, t_v, sem_t)
        cps = [None, None]
        cps[0] = pltpu.async_copy(x_hbm.at[pl.ds(xrow0, CROWS)], x_v0, sem0)
        t_cp.wait()
        iota = lax.iota(jnp.int32, LANES)
        UNR = 8
        for c in range(n_chunks):
            if c + 1 < n_chunks:
                nxt = (c + 1) % 2
                cps[nxt] = pltpu.async_copy(
                    x_hbm.at[pl.ds(xrow0 + (c + 1) * CROWS, CROWS)],
                    bufs[nxt],
                    sems[nxt],
                )
            cps[c % 2].wait()
            x_v = bufs[c % 2]
            for g in range(0, CHUNK // LANES, 2):
                offs_a = [(iota + g * LANES) * HIST + k for k in range(UNR)]
                offs_b = [(iota + (g + 1) * LANES) * HIST + k for k in range(UNR)]

                def jbody(i, carry, offs_a=offs_a, offs_b=offs_b, x_v=x_v):
                    acc_a, acc_b = carry
                    j = i * UNR
                    va = []
                    vb = []
                    for k in range(UNR):
                        pa = offs_a[k] + j
                        pb = offs_b[k] + j
                        ia = plsc.load_gather(x_v, [pa >> 7, pa & 127])
                        ib = plsc.load_gather(x_v, [pb >> 7, pb & 127])
                        va.append(plsc.load_gather(t_v, [ia]))
                        vb.append(plsc.load_gather(t_v, [ib]))
                    sa = ((va[0] + va[1]) + (va[2] + va[3])) + (
                        (va[4] + va[5]) + (va[6] + va[7])
                    )
                    sb = ((vb[0] + vb[1]) + (vb[2] + vb[3])) + (
                        (vb[4] + vb[5]) + (vb[6] + vb[7])
                    )
                    return acc_a + sa, acc_b + sb

                zero = jnp.zeros((LANES,), jnp.float32)
                acc_a, acc_b = lax.fori_loop(0, HIST // UNR, jbody, (zero, zero))
                s_v[pl.ds(c * CHUNK + g * LANES, LANES)] = acc_a
                s_v[pl.ds(c * CHUNK + (g + 1) * LANES, LANES)] = acc_b
        for i in range(ROWS_PER // LANES):
            sv = s_v[pl.ds(i * LANES, LANES)]
            p = 1.0 / (1.0 + jnp.exp(sv * (-1.0 / HIST)))
            yi = (p * 10000.0 + 0.5).astype(jnp.int32)
            s_v[pl.ds(i * LANES, LANES)] = yi.astype(jnp.float32) * (1.0 / 10000.0)
        pltpu.sync_copy(s_v, out_hbm.at[pl.ds(base, ROWS_PER)])

    return k(t, x_flat)


def kernel(x, table, W, b):
    m = jnp.kron(jnp.eye(8, dtype=jnp.float32), W.astype(jnp.float32))
    table_r = table.reshape(VOCAB // 8, 128)
    b2 = b.reshape(1, 1)

    t = _tc_table_w(table_r, m, b2).reshape(VOCAB)
    probs = _sc_gather_sum(t, x.reshape(BATCH * HIST // 128, 128))
    return probs.reshape(BATCH, 1)
